# Initial kernel scaffold; baseline (speedup 1.0000x reference)
import jax
import jax.numpy as jnp
from jax.experimental import pallas as pl


def kernel(x, edge_index, edge_attr, W_in0, W_r0, W_r1, W_r2, W_out0, W_out1, W_out2):
    raise NotImplementedError("write your pallas kernel here")


# speedup vs baseline: 2.2542x; 2.2542x over previous
import functools
import math

import jax
import jax.numpy as jnp
from jax import lax
from jax.experimental import pallas as pl
from jax.experimental.pallas import tpu as pltpu
from jax.experimental.pallas import tpu_sc as plsc

N = 10000
E = 160000
F = 128
B = 8
EPS = 1e-6

NC = 2
NS = 16
NW = NC * NS

E_PAD = 163840
CHUNK = 128
CH_PER_TILE = E_PAD // NW // CHUNK
N_ACC = 10240
ROWS_PER_TILE_ZERO = N_ACC // NS
ROWS_PER_TILE_OUT = N // NS

RSQRT_F = 1.0 / math.sqrt(float(F))
DEG_NORM = 1.0 / math.sqrt(float(E) / float(N))
SQRT3 = math.sqrt(3.0)
SQRT15 = math.sqrt(15.0)
SQRT5_2 = math.sqrt(5.0) / 2.0
INV2SIG2 = 1.0 / (2.0 * 0.35 ** 2)


def _h_body(x_ref, w_ref, o_ref):
    o_ref[...] = jnp.dot(x_ref[...], w_ref[...],
                         preferred_element_type=jnp.float32) * RSQRT_F


def _input_linear(x0, W_in0):
    blk = 1000
    return pl.pallas_call(
        _h_body,
        grid=(N // blk,),
        in_specs=[pl.BlockSpec((blk, F), lambda i: (i, 0)),
                  pl.BlockSpec((F, F), lambda i: (0, 0))],
        out_specs=pl.BlockSpec((blk, F), lambda i: (i, 0)),
        out_shape=jax.ShapeDtypeStruct((N, F), jnp.float32),
    )(x0, W_in0)


def _gather_body(h_hbm, idx_hbm, out_hbm, idx_v, rows_v, sem):
    c = lax.axis_index("c")
    s = lax.axis_index("s")
    base = (s * NC + c) * (CH_PER_TILE * CHUNK)

    def chunk(j, carry):
        e0 = base + j * CHUNK
        pltpu.sync_copy(idx_hbm.at[pl.ds(e0, CHUNK)], idx_v)
        pltpu.async_copy(h_hbm.at[idx_v], rows_v, sem).wait()
        pltpu.sync_copy(rows_v, out_hbm.at[pl.ds(e0, CHUNK)])
        return carry

    lax.fori_loop(0, CH_PER_TILE, chunk, 0)


def _gather(h, src_p):
    mesh = plsc.VectorSubcoreMesh(core_axis_name="c", subcore_axis_name="s",
                                  num_cores=NC, num_subcores=NS)
    fn = functools.partial(
        pl.kernel,
        out_type=jax.ShapeDtypeStruct((E_PAD, F), jnp.float32),
        mesh=mesh,
        scratch_types=[
            pltpu.VMEM((CHUNK,), jnp.int32),
            pltpu.VMEM((CHUNK, F), jnp.float32),
            pltpu.SemaphoreType.DMA,
        ],
    )(_gather_body)
    return fn(h, src_p)


def _msg_body(hs_ref, ea_ref, w0_ref, w1_ref, w2_ref, o_ref):
    hs = hs_ref[...]
    ea = ea_ref[...]
    ex, ey, ez = ea[:, 0:1], ea[:, 1:2], ea[:, 2:3]
    d = jnp.sqrt(ex * ex + ey * ey + ez * ez + EPS)
    inv_d = 1.0 / d
    ux, uy, uz = ex * inv_d, ey * inv_d, ez * inv_d

    parts = []
    for b in range(B):
        cb = 2.5 * b / (B - 1)
        basis_b = jnp.exp(-((d - cb) ** 2) * INV2SIG2)
        parts.append(basis_b * hs)
    hb = jnp.concatenate(parts, axis=1)

    s0 = jnp.dot(hb, w0_ref[...], preferred_element_type=jnp.float32) * RSQRT_F
    s1 = jnp.dot(hb, w1_ref[...], preferred_element_type=jnp.float32) * RSQRT_F
    s2 = jnp.dot(hb, w2_ref[...], preferred_element_type=jnp.float32) * RSQRT_F

    y1 = (SQRT3 * ux, SQRT3 * uy, SQRT3 * uz)
    y2 = (SQRT15 * ux * uy,
          SQRT15 * uy * uz,
          SQRT5_2 * (3.0 * uz * uz - 1.0),
          SQRT15 * ux * uz,
          (SQRT15 / 2.0) * (ux * ux - uy * uy))

    o_ref[0] = s0
    for m in range(3):
        o_ref[1 + m] = s1 * y1[m]
    for m in range(5):
        o_ref[4 + m] = s2 * y2[m]


def _edge_messages(hs, ea_p, Wr0f, Wr1f, Wr2f):
    blk = 512
    return pl.pallas_call(
        _msg_body,
        grid=(E_PAD // blk,),
        in_specs=[pl.BlockSpec((blk, F), lambda i: (i, 0)),
                  pl.BlockSpec((blk, 3), lambda i: (i, 0)),
                  pl.BlockSpec((B * F, F), lambda i: (0, 0)),
                  pl.BlockSpec((B * F, F), lambda i: (0, 0)),
                  pl.BlockSpec((B * F, F), lambda i: (0, 0))],
        out_specs=pl.BlockSpec((9, blk, F), lambda i: (0, i, 0)),
        out_shape=jax.ShapeDtypeStruct((9, E_PAD, F), jnp.float32),
    )(hs, ea_p, Wr0f, Wr1f, Wr2f)


def _scatter_body(msg_hbm, dst_hbm, out_hbm, idx_v, msg_v, zero_v, acc, sem):
    c = lax.axis_index("c")
    s = lax.axis_index("s")

    def zb(k, carry):
        zero_v[k // 8, pl.ds((k % 8) * 16, 16)] = jnp.zeros((16,), jnp.float32)
        return carry
    lax.fori_loop(0, CHUNK * F // 16, zb, 0)

    for p in range(9):
        for j in range(ROWS_PER_TILE_ZERO // CHUNK):
            r0 = s * ROWS_PER_TILE_ZERO + j * CHUNK
            pltpu.sync_copy(zero_v, acc.at[pl.ds(r0, CHUNK)])
        plsc.subcore_barrier()

        def chunk(j, carry):
            e0 = c * (E_PAD // NC) + s * (CH_PER_TILE * CHUNK) + j * CHUNK
            pltpu.sync_copy(dst_hbm.at[pl.ds(e0, CHUNK)], idx_v)
            pltpu.sync_copy(msg_hbm.at[p, pl.ds(e0, CHUNK)], msg_v)
            pltpu.sync_copy(msg_v, acc.at[idx_v], add=True)
            return carry
        lax.fori_loop(0, CH_PER_TILE, chunk, 0)
        plsc.subcore_barrier()

        for j in range(ROWS_PER_TILE_ZERO // CHUNK):
            r0 = s * ROWS_PER_TILE_ZERO + j * CHUNK
            pltpu.sync_copy(acc.at[pl.ds(r0, CHUNK)], msg_v)
            pltpu.sync_copy(msg_v, out_hbm.at[p, c, pl.ds(r0, CHUNK)])
        plsc.subcore_barrier()


def _scatter(msg, dst_p):
    mesh = plsc.VectorSubcoreMesh(core_axis_name="c", subcore_axis_name="s",
                                  num_cores=NC, num_subcores=NS)
    fn = functools.partial(
        pl.kernel,
        out_type=jax.ShapeDtypeStruct((9, NC, N_ACC, F), jnp.float32),
        mesh=mesh,
        scratch_types=[
            pltpu.VMEM((CHUNK,), jnp.int32),
            pltpu.VMEM((CHUNK, F), jnp.float32),
            pltpu.VMEM((CHUNK, F), jnp.float32),
            pltpu.VMEM_SHARED((N_ACC, F), jnp.float32),
            pltpu.SemaphoreType.DMA,
        ],
    )(_scatter_body)
    return fn(msg, dst_p)


def _out_body(part_ref, w0_ref, w1_ref, w2_ref, o0_ref, o1_ref, o2_ref):
    pr = part_ref[...]
    g = (pr[:, 0] + pr[:, 1]) * DEG_NORM

    a0 = g[0]
    a1 = [g[1 + m] for m in range(3)]
    a2 = [g[4 + m] for m in range(5)]

    rms0 = jnp.sqrt(jnp.mean(a0 * a0, axis=-1, keepdims=True) + EPS)
    n0 = a0 / rms0
    ss1 = sum(jnp.sum(t * t, axis=-1, keepdims=True) for t in a1)
    rms1 = jnp.sqrt(ss1 / (3.0 * F) + EPS)
    ss2 = sum(jnp.sum(t * t, axis=-1, keepdims=True) for t in a2)
    rms2 = jnp.sqrt(ss2 / (5.0 * F) + EPS)

    o0 = jnp.dot(n0, w0_ref[...], preferred_element_type=jnp.float32) * RSQRT_F
    o0_ref[...] = jax.nn.relu(o0)

    t1 = [jnp.dot(t / rms1, w1_ref[...], preferred_element_type=jnp.float32)
          * RSQRT_F for t in a1]
    nn1 = jnp.sqrt(sum(t * t for t in t1) + EPS)
    f1 = nn1 / (nn1 + EPS)
    o1_ref[...] = jnp.concatenate([t * f1 for t in t1], axis=1)

    t2 = [jnp.dot(t / rms2, w2_ref[...], preferred_element_type=jnp.float32)
          * RSQRT_F for t in a2]
    nn2 = jnp.sqrt(sum(t * t for t in t2) + EPS)
    f2 = nn2 / (nn2 + EPS)
    o2_ref[...] = jnp.concatenate([t * f2 for t in t2], axis=1)


def _node_epilogue(part, W_out0, W_out1, W_out2):
    blk = 200
    return pl.pallas_call(
        _out_body,
        grid=(N // blk,),
        in_specs=[pl.BlockSpec((9, NC, blk, F), lambda i: (0, 0, i, 0)),
                  pl.BlockSpec((F, F), lambda i: (0, 0)),
                  pl.BlockSpec((F, F), lambda i: (0, 0)),
                  pl.BlockSpec((F, F), lambda i: (0, 0))],
        out_specs=[pl.BlockSpec((blk, F), lambda i: (i, 0)),
                   pl.BlockSpec((blk, 3 * F), lambda i: (i, 0)),
                   pl.BlockSpec((blk, 5 * F), lambda i: (i, 0))],
        out_shape=[jax.ShapeDtypeStruct((N, F), jnp.float32),
                   jax.ShapeDtypeStruct((N, 3 * F), jnp.float32),
                   jax.ShapeDtypeStruct((N, 5 * F), jnp.float32)],
    )(part, W_out0, W_out1, W_out2)


def kernel(x, edge_index, edge_attr, W_in0, W_r0, W_r1, W_r2,
           W_out0, W_out1, W_out2):
    x0 = x[0]
    src = edge_index[0]
    dst = edge_index[1]
    pad = E_PAD - E
    src_p = jnp.concatenate([src, jnp.zeros((pad,), jnp.int32)])
    dst_p = jnp.concatenate([dst, jnp.full((pad,), N, jnp.int32)])
    ea_p = jnp.concatenate([edge_attr, jnp.zeros((pad, 3), jnp.float32)])

    Wr0f = W_r0.transpose(0, 2, 1).reshape(B * F, F)
    Wr1f = W_r1.transpose(0, 2, 1).reshape(B * F, F)
    Wr2f = W_r2.transpose(0, 2, 1).reshape(B * F, F)

    h = _input_linear(x0, W_in0)
    hs = _gather(h, src_p)
    msg = _edge_messages(hs, ea_p, Wr0f, Wr1f, Wr2f)
    part = _scatter(msg, dst_p)
    o0, o1, o2 = _node_epilogue(part, W_out0, W_out1, W_out2)

    out1 = o1.reshape(N, 3, F).transpose(0, 2, 1).reshape(N, 3 * F)
    out2 = o2.reshape(N, 5, F).transpose(0, 2, 1).reshape(N, 5 * F)
    return (o0, out1, out2)
